# baseline (speedup 1.0000x reference)
import jax
import jax.numpy as jnp
from jax.experimental import pallas as pl

_NUM_SCALES = 1000
_TPAD = 1024


def _body(t_ref, tab_ref, x_ref, n_ref, o_ref):
    tcol = t_ref[...][:, None]
    br = tcol.shape[0]
    hi = tcol >> 7
    lo = tcol & 127
    oh_hi = (hi == jax.lax.broadcasted_iota(jnp.int32, (br, 8), 1)).astype(jnp.float32)
    oh_lo = lo == jax.lax.broadcasted_iota(jnp.int32, (br, 128), 1)
    rows = jnp.dot(oh_hi, tab_ref[...], preferred_element_type=jnp.float32)
    a = jnp.sum(jnp.where(oh_lo, rows[:, :128], 0.0), axis=1, keepdims=True)
    b = jnp.sum(jnp.where(oh_lo, rows[:, 128:], 0.0), axis=1, keepdims=True)
    o_ref[...] = a * x_ref[...] + b * n_ref[...]


def kernel(x_start, t, noise, sqrt_alphas_cumprod, sqrt_one_minus_alphas_cumprod):
    batch, dim = x_start.shape
    br = 8192
    nb = batch // br
    t1 = t.astype(jnp.int32)
    taba = jnp.zeros((_TPAD,), jnp.float32).at[:_NUM_SCALES].set(
        sqrt_alphas_cumprod).reshape(8, 128)
    tabb = jnp.zeros((_TPAD,), jnp.float32).at[:_NUM_SCALES].set(
        sqrt_one_minus_alphas_cumprod).reshape(8, 128)
    tab = jnp.concatenate([taba, tabb], axis=1)
    return pl.pallas_call(
        _body,
        grid=(nb,),
        in_specs=[
            pl.BlockSpec((br,), lambda i: (i,)),
            pl.BlockSpec((8, 256), lambda i: (0, 0)),
            pl.BlockSpec((br, dim), lambda i: (i, 0)),
            pl.BlockSpec((br, dim), lambda i: (i, 0)),
        ],
        out_specs=pl.BlockSpec((br, dim), lambda i: (i, 0)),
        out_shape=jax.ShapeDtypeStruct((batch, dim), jnp.float32),
    )(t1, tab, x_start, noise)


# speedup vs baseline: 1.0477x; 1.0477x over previous
import jax
import jax.numpy as jnp
from jax.experimental import pallas as pl

_NUM_SCALES = 1000
_TPAD = 1024


def _body(t_ref, tab_ref, x_ref, n_ref, o_ref):
    tcol = t_ref[...][:, None]
    br = tcol.shape[0]
    hi = tcol >> 7
    lo = tcol & 127
    oh_hi = (hi == jax.lax.broadcasted_iota(jnp.int32, (br, 8), 1)).astype(jnp.bfloat16)
    oh_lo = lo == jax.lax.broadcasted_iota(jnp.int32, (br, 128), 1)
    rows = jnp.dot(oh_hi, tab_ref[...], preferred_element_type=jnp.float32)
    a = jnp.sum(jnp.where(oh_lo, rows[:, :128], 0.0), axis=1, keepdims=True)
    b = jnp.sum(jnp.where(oh_lo, rows[:, 128:], 0.0), axis=1, keepdims=True)
    o_ref[...] = a * x_ref[...] + b * n_ref[...]


def kernel(x_start, t, noise, sqrt_alphas_cumprod, sqrt_one_minus_alphas_cumprod):
    batch, dim = x_start.shape
    br = 4096
    nb = batch // br
    t1 = t.astype(jnp.int32)
    taba = jnp.zeros((_TPAD,), jnp.float32).at[:_NUM_SCALES].set(
        sqrt_alphas_cumprod).reshape(8, 128)
    tabb = jnp.zeros((_TPAD,), jnp.float32).at[:_NUM_SCALES].set(
        sqrt_one_minus_alphas_cumprod).reshape(8, 128)
    tab = jnp.concatenate([taba, tabb], axis=1).astype(jnp.bfloat16)
    return pl.pallas_call(
        _body,
        grid=(nb,),
        in_specs=[
            pl.BlockSpec((br,), lambda i: (i,)),
            pl.BlockSpec((8, 256), lambda i: (0, 0)),
            pl.BlockSpec((br, dim), lambda i: (i, 0)),
            pl.BlockSpec((br, dim), lambda i: (i, 0)),
        ],
        out_specs=pl.BlockSpec((br, dim), lambda i: (i, 0)),
        out_shape=jax.ShapeDtypeStruct((batch, dim), jnp.float32),
    )(t1, tab, x_start, noise)


# speedup vs baseline: 1.0867x; 1.0372x over previous
import jax
import jax.numpy as jnp
from jax.experimental import pallas as pl

_NUM_SCALES = 1000
_TPAD = 1024


def _body(t_ref, tabs_ref, x_ref, n_ref, o_ref):
    tcol = t_ref[...][:, None]
    br = tcol.shape[0]
    hi = tcol >> 7
    lo = tcol & 127
    t16 = tabs_ref[...].reshape(16, 128)
    tab = jnp.concatenate([t16[0:8], t16[8:16]], axis=1)
    oh_hi = (hi == jax.lax.broadcasted_iota(jnp.int32, (br, 8), 1)).astype(jnp.float32)
    oh_lo = lo == jax.lax.broadcasted_iota(jnp.int32, (br, 128), 1)
    rows = jnp.dot(oh_hi, tab, preferred_element_type=jnp.float32)
    a = jnp.sum(jnp.where(oh_lo, rows[:, :128], 0.0), axis=1, keepdims=True)
    b = jnp.sum(jnp.where(oh_lo, rows[:, 128:], 0.0), axis=1, keepdims=True)
    o_ref[...] = a * x_ref[...] + b * n_ref[...]


def kernel(x_start, t, noise, sqrt_alphas_cumprod, sqrt_one_minus_alphas_cumprod):
    batch, dim = x_start.shape
    br = 4096
    nb = batch // br
    t1 = t.astype(jnp.int32)
    pad = jnp.zeros((_TPAD - _NUM_SCALES,), jnp.float32)
    tabs = jnp.concatenate(
        [sqrt_alphas_cumprod, pad, sqrt_one_minus_alphas_cumprod, pad])
    return pl.pallas_call(
        _body,
        grid=(nb,),
        in_specs=[
            pl.BlockSpec((br,), lambda i: (i,)),
            pl.BlockSpec((2 * _TPAD,), lambda i: (0,)),
            pl.BlockSpec((br, dim), lambda i: (i, 0)),
            pl.BlockSpec((br, dim), lambda i: (i, 0)),
        ],
        out_specs=pl.BlockSpec((br, dim), lambda i: (i, 0)),
        out_shape=jax.ShapeDtypeStruct((batch, dim), jnp.float32),
    )(t1, tabs, x_start, noise)


# speedup vs baseline: 1.0927x; 1.0056x over previous
import jax
import jax.numpy as jnp
from jax.experimental import pallas as pl

_NUM_SCALES = 1000
_TPAD = 1024


def _body(t_ref, tab_ref, x_ref, n_ref, o_ref):
    tcol = t_ref[...][:, None]
    br = tcol.shape[0]
    hi = tcol >> 7
    lo = tcol & 127
    oh_hi = (hi == jax.lax.broadcasted_iota(jnp.int32, (br, 8), 1)).astype(jnp.float32)
    oh_lo = lo == jax.lax.broadcasted_iota(jnp.int32, (br, 128), 1)
    rows = jnp.dot(oh_hi, tab_ref[...], preferred_element_type=jnp.float32)
    a = jnp.sum(jnp.where(oh_lo, rows[:, :128], 0.0), axis=1, keepdims=True)
    b = jnp.sum(jnp.where(oh_lo, rows[:, 128:], 0.0), axis=1, keepdims=True)
    o_ref[...] = a * x_ref[...] + b * n_ref[...]


def kernel(x_start, t, noise, sqrt_alphas_cumprod, sqrt_one_minus_alphas_cumprod):
    batch, dim = x_start.shape
    br = 4096
    nb = batch // br
    t1 = t.astype(jnp.int32)
    taba = jnp.zeros((_TPAD,), jnp.float32).at[:_NUM_SCALES].set(
        sqrt_alphas_cumprod).reshape(8, 128)
    tabb = jnp.zeros((_TPAD,), jnp.float32).at[:_NUM_SCALES].set(
        sqrt_one_minus_alphas_cumprod).reshape(8, 128)
    tab = jnp.concatenate([taba, tabb], axis=1)
    return pl.pallas_call(
        _body,
        grid=(nb,),
        in_specs=[
            pl.BlockSpec((br,), lambda i: (i,)),
            pl.BlockSpec((8, 256), lambda i: (0, 0)),
            pl.BlockSpec((br, dim), lambda i: (i, 0)),
            pl.BlockSpec((br, dim), lambda i: (i, 0)),
        ],
        out_specs=pl.BlockSpec((br, dim), lambda i: (i, 0)),
        out_shape=jax.ShapeDtypeStruct((batch, dim), jnp.float32),
    )(t1, tab, x_start, noise)
